# Initial kernel scaffold; baseline (speedup 1.0000x reference)
import jax
import jax.numpy as jnp
from jax.experimental import pallas as pl


def kernel(x, W1, W2, fc_W, fc_b):
    raise NotImplementedError("write your pallas kernel here")


# speedup vs baseline: 153.5076x; 153.5076x over previous
import functools

import jax
import jax.numpy as jnp
from jax import lax
from jax.experimental import pallas as pl
from jax.experimental.pallas import tpu as pltpu
from jax.experimental.pallas import tpu_sc as plsc

N = 8192
K = 8
N_BLADES = 8
N_FREE = 5
DT = 0.1

ROWS = 256
CHUNK = 128
TOPL = 5
IMAX = 2**31 - 1
IMIN = -2**31

NC = 2
NS = 16
NW = NC * NS
NODES_W = N // NW
EDGES_W = NODES_W * K
IDX_ROWS = EDGES_W // 128


def _knn_kernel(xr_ref, xct_ref, sqr_ref, sqc_ref, o_ref):
    i = pl.program_id(0)
    lane = lax.broadcasted_iota(jnp.int32, (ROWS, CHUNK), 1)
    sqr = sqr_ref[...]
    m = [jnp.full((ROWS, CHUNK), jnp.inf, jnp.float32) for _ in range(TOPL)]
    for c in range(N // CHUNK):
        dot = jnp.dot(xr_ref[...], xct_ref[:, c * CHUNK:(c + 1) * CHUNK],
                      preferred_element_type=jnp.float32)
        d2 = (sqr + sqc_ref[:, c * CHUNK:(c + 1) * CHUNK]) + dot
        bits = lax.bitcast_convert_type(jnp.maximum(d2, 1e-30), jnp.int32)
        key = lax.bitcast_convert_type((bits & (-64)) | c, jnp.float32)
        t = key
        for j in range(TOPL):
            nm = jnp.minimum(m[j], t)
            t = jnp.maximum(m[j], t)
            m[j] = nm
    prev = jnp.full((ROWS, 1), -jnp.inf, jnp.float32)
    idxs = []
    for _ in range(K + 1):
        t = jnp.full((ROWS, CHUNK), jnp.inf, jnp.float32)
        for mm in m:
            t = jnp.minimum(t, jnp.where(mm > prev, mm, jnp.inf))
        kj = jnp.min(t, axis=1, keepdims=True)
        lane_j = jnp.min(jnp.where(t == kj, lane, IMAX), axis=1,
                         keepdims=True)
        cj = (lax.bitcast_convert_type(kj, jnp.int32) & 63) * CHUNK + lane_j
        idxs.append(cj)
        prev = kj
    rows = lax.broadcasted_iota(jnp.int32, (ROWS, 1), 0) + i * ROWS
    cum = idxs[0] == rows
    outs = []
    for s in range(K):
        cum = cum | (idxs[s] == rows) if s else cum
        outs.append(jnp.where(cum, idxs[s + 1], idxs[s]))
    o_ref[...] = jnp.concatenate(outs, axis=1)


def _knn(x0):
    sq = jnp.sum(x0 * x0, axis=-1)
    zeros5 = jnp.zeros((N, 5), x0.dtype)
    xr = jnp.concatenate([x0, zeros5], axis=1)
    xct = jnp.concatenate([-2.0 * x0, zeros5], axis=1).T
    return pl.pallas_call(
        _knn_kernel,
        grid=(N // ROWS,),
        in_specs=[
            pl.BlockSpec((ROWS, 8), lambda i: (i, 0)),
            pl.BlockSpec((8, N), lambda i: (0, 0)),
            pl.BlockSpec((ROWS, 1), lambda i: (i, 0)),
            pl.BlockSpec((1, N), lambda i: (0, 0)),
        ],
        out_specs=pl.BlockSpec((ROWS, K), lambda i: (i, 0)),
        out_shape=jax.ShapeDtypeStruct((N, K), jnp.int32),
    )(xr, xct, sq[:, None], sq[None, :])


def _sc_gather_body(h_hbm, col_hbm, out_hbm, idx_v, rows_v, own_v, res_v, sem):
    wid = lax.axis_index("s") * NC + lax.axis_index("c")
    nbase = wid * NODES_W
    pltpu.sync_copy(col_hbm.at[pl.ds(wid * IDX_ROWS, IDX_ROWS)], idx_v)
    copies = [
        pltpu.async_copy(h_hbm.at[idx_v.at[j]],
                         rows_v.at[pl.ds(j * 128, 128)], sem)
        for j in range(IDX_ROWS)
    ]
    pltpu.sync_copy(h_hbm.at[pl.ds(nbase, NODES_W)], own_v)
    for cp in copies:
        cp.wait()

    def body(n, _):
        for half in (0, 16):
            acc = rows_v[n * K, pl.ds(half, 16)]
            for kk in range(1, K):
                acc = acc + rows_v[n * K + kk, pl.ds(half, 16)]
            res_v[n, pl.ds(half, 16)] = (
                own_v[n, pl.ds(half, 16)] + (1.0 / K) * acc)
        return 0

    lax.fori_loop(0, NODES_W, body, 0)
    pltpu.sync_copy(res_v, out_hbm.at[pl.ds(nbase, NODES_W)])


@functools.cache
def _sc_gather():
    mesh = plsc.VectorSubcoreMesh(
        core_axis_name="c", subcore_axis_name="s",
        num_cores=NC, num_subcores=NS)
    return pl.kernel(
        _sc_gather_body,
        out_type=jax.ShapeDtypeStruct((N, 32), jnp.float32),
        mesh=mesh,
        scratch_types=[
            pltpu.VMEM((IDX_ROWS, 128), jnp.int32),
            pltpu.VMEM((EDGES_W, 32), jnp.float32),
            pltpu.VMEM((NODES_W, 32), jnp.float32),
            pltpu.VMEM((NODES_W, 32), jnp.float32),
            pltpu.SemaphoreType.DMA,
        ],
        compiler_params=pltpu.CompilerParams(use_tc_tiling_on_sc=False),
    )


UROWS = 2048


def _update_kernel(g_ref, w_ref, h_ref, o_ref):
    o_ref[...] = h_ref[...] + DT * jnp.tanh(
        jnp.dot(g_ref[...], w_ref[...], preferred_element_type=jnp.float32))


def _tc_update(g, wblk, h):
    return pl.pallas_call(
        _update_kernel,
        grid=(N // UROWS,),
        in_specs=[
            pl.BlockSpec((UROWS, 32), lambda i: (i, 0)),
            pl.BlockSpec((32, 32), lambda i: (0, 0)),
            pl.BlockSpec((UROWS, 32), lambda i: (i, 0)),
        ],
        out_specs=pl.BlockSpec((UROWS, 32), lambda i: (i, 0)),
        out_shape=jax.ShapeDtypeStruct((N, 32), jnp.float32),
    )(g, wblk, h)


def _head_kernel(h_ref, w2_ref, fc_ref, b_ref, o_ref):
    pooled = jnp.mean(h_ref[...], axis=0, keepdims=True)
    h2 = pooled
    for _ in range(N_FREE):
        h2 = h2 + DT * jnp.tanh(
            jnp.dot(2.0 * h2, w2_ref[...], preferred_element_type=jnp.float32))
    o_ref[...] = jnp.dot(h2, fc_ref[...],
                         preferred_element_type=jnp.float32) + b_ref[...]


def _head(h, w2blk, fcblk, bias):
    return pl.pallas_call(
        _head_kernel,
        in_specs=[
            pl.BlockSpec((N, 32), lambda: (0, 0)),
            pl.BlockSpec((32, 32), lambda: (0, 0)),
            pl.BlockSpec((32, 16), lambda: (0, 0)),
            pl.BlockSpec((1, 16), lambda: (0, 0)),
        ],
        out_specs=pl.BlockSpec((1, 16), lambda: (0, 0)),
        out_shape=jax.ShapeDtypeStruct((1, 16), jnp.float32),
    )(h, w2blk, fcblk, bias)


def kernel(x, W1, W2, fc_W, fc_b):
    B, n, _ = x.shape
    x0 = x[0]
    nbr = _knn(x0)
    col = nbr.reshape(NW * IDX_ROWS, 128)

    xt = jnp.transpose(x, (1, 0, 2))
    h0 = jnp.zeros((n, B, N_BLADES), jnp.float32).at[:, :, 1:4].set(xt)
    h = h0.reshape(n, B * N_BLADES)

    eye = jnp.eye(B, dtype=jnp.float32)
    w1blk = jnp.kron(eye, W1)
    w2blk = jnp.kron(eye, W2)
    fcblk = jnp.kron(eye, fc_W)
    bias = jnp.tile(fc_b, B)[None, :]

    gather = _sc_gather()
    for _ in range(N_FREE):
        g = gather(h, col)
        h = _tc_update(g, w1blk, h)

    logits = _head(h, w2blk, fcblk, bias)
    return logits.reshape(B, 4)


# speedup vs baseline: 166.5337x; 1.0849x over previous
import functools

import jax
import jax.numpy as jnp
from jax import lax
from jax.experimental import pallas as pl
from jax.experimental.pallas import tpu as pltpu
from jax.experimental.pallas import tpu_sc as plsc

N = 8192
K = 8
N_BLADES = 8
N_FREE = 5
DT = 0.1

ROWS = 256
CHUNK = 128
TOPL = 4
IMAX = 2**31 - 1
IMIN = -2**31

NC = 2
NS = 16
NW = NC * NS
NODES_W = N // NW
EDGES_W = NODES_W * K
IDX_ROWS = EDGES_W // 128


def _knn_kernel(xr_ref, xct_ref, sqr_ref, sqc_ref, o_ref):
    i = pl.program_id(0)
    lane = lax.broadcasted_iota(jnp.int32, (ROWS, CHUNK), 1)
    sqr = sqr_ref[...]
    m = [jnp.full((ROWS, CHUNK), jnp.inf, jnp.float32) for _ in range(TOPL)]
    for c in range(N // CHUNK):
        dot = jnp.dot(xr_ref[...], xct_ref[:, c * CHUNK:(c + 1) * CHUNK],
                      preferred_element_type=jnp.float32)
        d2 = (sqr + sqc_ref[:, c * CHUNK:(c + 1) * CHUNK]) + dot
        bits = lax.bitcast_convert_type(jnp.maximum(d2, 1e-30), jnp.int32)
        key = lax.bitcast_convert_type((bits & (-64)) | c, jnp.float32)
        t = key
        for j in range(TOPL):
            nm = jnp.minimum(m[j], t)
            t = jnp.maximum(m[j], t)
            m[j] = nm
    prev = jnp.full((ROWS, 1), -jnp.inf, jnp.float32)
    idxs = []
    for _ in range(K + 1):
        t = jnp.full((ROWS, CHUNK), jnp.inf, jnp.float32)
        for mm in m:
            t = jnp.minimum(t, jnp.where(mm > prev, mm, jnp.inf))
        kj = jnp.min(t, axis=1, keepdims=True)
        lane_j = jnp.min(jnp.where(t == kj, lane, IMAX), axis=1,
                         keepdims=True)
        cj = (lax.bitcast_convert_type(kj, jnp.int32) & 63) * CHUNK + lane_j
        idxs.append(cj)
        prev = kj
    rows = lax.broadcasted_iota(jnp.int32, (ROWS, 1), 0) + i * ROWS
    cum = idxs[0] == rows
    outs = []
    for s in range(K):
        cum = cum | (idxs[s] == rows) if s else cum
        outs.append(jnp.where(cum, idxs[s + 1], idxs[s]))
    o_ref[...] = jnp.concatenate(outs, axis=1)


def _knn(x0):
    sq = jnp.sum(x0 * x0, axis=-1)
    zeros5 = jnp.zeros((N, 5), x0.dtype)
    xr = jnp.concatenate([x0, zeros5], axis=1)
    xct = jnp.concatenate([-2.0 * x0, zeros5], axis=1).T
    return pl.pallas_call(
        _knn_kernel,
        grid=(N // ROWS,),
        in_specs=[
            pl.BlockSpec((ROWS, 8), lambda i: (i, 0)),
            pl.BlockSpec((8, N), lambda i: (0, 0)),
            pl.BlockSpec((ROWS, 1), lambda i: (i, 0)),
            pl.BlockSpec((1, N), lambda i: (0, 0)),
        ],
        out_specs=pl.BlockSpec((ROWS, K), lambda i: (i, 0)),
        out_shape=jax.ShapeDtypeStruct((N, K), jnp.int32),
    )(xr, xct, sq[:, None], sq[None, :])


def _sc_gather_body(h_hbm, col_hbm, out_hbm, idx_v, rows_v, own_v, res_v, sem):
    wid = lax.axis_index("s") * NC + lax.axis_index("c")
    nbase = wid * NODES_W
    pltpu.sync_copy(col_hbm.at[pl.ds(wid * IDX_ROWS, IDX_ROWS)], idx_v)
    copies = [
        pltpu.async_copy(h_hbm.at[idx_v.at[j]],
                         rows_v.at[pl.ds(j * 128, 128)], sem)
        for j in range(IDX_ROWS)
    ]
    pltpu.sync_copy(h_hbm.at[pl.ds(nbase, NODES_W)], own_v)
    for cp in copies:
        cp.wait()

    def body(nn, _):
        for u in range(4):
            n = nn * 4 + u
            for half in (0, 16):
                acc0 = rows_v[n * K, pl.ds(half, 16)] + \
                    rows_v[n * K + 1, pl.ds(half, 16)]
                acc1 = rows_v[n * K + 2, pl.ds(half, 16)] + \
                    rows_v[n * K + 3, pl.ds(half, 16)]
                acc2 = rows_v[n * K + 4, pl.ds(half, 16)] + \
                    rows_v[n * K + 5, pl.ds(half, 16)]
                acc3 = rows_v[n * K + 6, pl.ds(half, 16)] + \
                    rows_v[n * K + 7, pl.ds(half, 16)]
                acc = (acc0 + acc1) + (acc2 + acc3)
                res_v[n, pl.ds(half, 16)] = (
                    own_v[n, pl.ds(half, 16)] + (1.0 / K) * acc)
        return 0

    lax.fori_loop(0, NODES_W // 4, body, 0)
    pltpu.sync_copy(res_v, out_hbm.at[pl.ds(nbase, NODES_W)])


@functools.cache
def _sc_gather():
    mesh = plsc.VectorSubcoreMesh(
        core_axis_name="c", subcore_axis_name="s",
        num_cores=NC, num_subcores=NS)
    return pl.kernel(
        _sc_gather_body,
        out_type=jax.ShapeDtypeStruct((N, 32), jnp.float32),
        mesh=mesh,
        scratch_types=[
            pltpu.VMEM((IDX_ROWS, 128), jnp.int32),
            pltpu.VMEM((EDGES_W, 32), jnp.float32),
            pltpu.VMEM((NODES_W, 32), jnp.float32),
            pltpu.VMEM((NODES_W, 32), jnp.float32),
            pltpu.SemaphoreType.DMA,
        ],
        compiler_params=pltpu.CompilerParams(use_tc_tiling_on_sc=False),
    )


UROWS = 2048


def _update_kernel(g_ref, w_ref, h_ref, o_ref):
    o_ref[...] = h_ref[...] + DT * jnp.tanh(
        jnp.dot(g_ref[...], w_ref[...], preferred_element_type=jnp.float32))


def _tc_update(g, wblk, h):
    return pl.pallas_call(
        _update_kernel,
        grid=(N // UROWS,),
        in_specs=[
            pl.BlockSpec((UROWS, 32), lambda i: (i, 0)),
            pl.BlockSpec((32, 32), lambda i: (0, 0)),
            pl.BlockSpec((UROWS, 32), lambda i: (i, 0)),
        ],
        out_specs=pl.BlockSpec((UROWS, 32), lambda i: (i, 0)),
        out_shape=jax.ShapeDtypeStruct((N, 32), jnp.float32),
    )(g, wblk, h)


def _head_kernel(h_ref, w2_ref, fc_ref, b_ref, o_ref):
    pooled = jnp.mean(h_ref[...], axis=0, keepdims=True)
    h2 = pooled
    for _ in range(N_FREE):
        h2 = h2 + DT * jnp.tanh(
            jnp.dot(2.0 * h2, w2_ref[...], preferred_element_type=jnp.float32))
    o_ref[...] = jnp.dot(h2, fc_ref[...],
                         preferred_element_type=jnp.float32) + b_ref[...]


def _head(h, w2blk, fcblk, bias):
    return pl.pallas_call(
        _head_kernel,
        in_specs=[
            pl.BlockSpec((N, 32), lambda: (0, 0)),
            pl.BlockSpec((32, 32), lambda: (0, 0)),
            pl.BlockSpec((32, 16), lambda: (0, 0)),
            pl.BlockSpec((1, 16), lambda: (0, 0)),
        ],
        out_specs=pl.BlockSpec((1, 16), lambda: (0, 0)),
        out_shape=jax.ShapeDtypeStruct((1, 16), jnp.float32),
    )(h, w2blk, fcblk, bias)


def kernel(x, W1, W2, fc_W, fc_b):
    B, n, _ = x.shape
    x0 = x[0]
    nbr = _knn(x0)
    col = nbr.reshape(NW * IDX_ROWS, 128)

    xt = jnp.transpose(x, (1, 0, 2))
    h0 = jnp.zeros((n, B, N_BLADES), jnp.float32).at[:, :, 1:4].set(xt)
    h = h0.reshape(n, B * N_BLADES)

    eye = jnp.eye(B, dtype=jnp.float32)
    w1blk = jnp.kron(eye, W1)
    w2blk = jnp.kron(eye, W2)
    fcblk = jnp.kron(eye, fc_W)
    bias = jnp.tile(fc_b, B)[None, :]

    gather = _sc_gather()
    for _ in range(N_FREE):
        g = gather(h, col)
        h = _tc_update(g, w1blk, h)

    logits = _head(h, w2blk, fcblk, bias)
    return logits.reshape(B, 4)
